# baseline (speedup 1.0000x reference)
import jax
import jax.numpy as jnp
from jax.experimental import pallas as pl
from jax.experimental.pallas import tpu as pltpu


def _dma_copy_body(t_ref, w_ref, ot_ref, ow_ref, sem_t, sem_w):
    ct = pltpu.make_async_copy(t_ref, ot_ref, sem_t)
    cw = pltpu.make_async_copy(w_ref, ow_ref, sem_w)
    ct.start()
    cw.start()
    ct.wait()
    cw.wait()


def kernel(bbox_targets, bbox_weights, labels):
    M, N = bbox_targets.shape
    out_t, out_w = pl.pallas_call(
        _dma_copy_body,
        in_specs=[pl.BlockSpec(memory_space=pl.ANY)] * 2,
        out_specs=[pl.BlockSpec(memory_space=pl.ANY)] * 2,
        out_shape=[jax.ShapeDtypeStruct((M, N), jnp.float32)] * 2,
        scratch_shapes=[pltpu.SemaphoreType.DMA] * 2,
    )(bbox_targets, bbox_weights)
    return out_t, out_w
